# Initial kernel scaffold; baseline (speedup 1.0000x reference)
import jax
import jax.numpy as jnp
from jax.experimental import pallas as pl


def kernel(k_cache, v_cache, input_pos, k_val, v_val):
    raise NotImplementedError("write your pallas kernel here")


# speedup vs baseline: 2.5930x; 2.5930x over previous
import jax
import jax.numpy as jnp
from jax.experimental import pallas as pl
from jax.experimental.pallas import tpu as pltpu

N_KV_HEADS = 8
HEAD_DIM = 128
MAX_SEQ_LEN = 8192
Q_LEN = 16

BS = 2048
NB = MAX_SEQ_LEN // BS


def _update_body(pos_ref, kc_ref, vc_ref, kval_ref, vval_ref, ko_ref, vo_ref):
    base = pl.program_id(1) * BS
    ko_ref[...] = kc_ref[...]
    vo_ref[...] = vc_ref[...]
    for i in range(Q_LEN):
        p = pos_ref[i]
        rel = p - base

        @pl.when((p >= base) & (p < base + BS))
        def _():
            ko_ref[0, pl.ds(rel, 1), :] = kval_ref[0, pl.ds(i, 1), :]
            vo_ref[0, pl.ds(rel, 1), :] = vval_ref[0, pl.ds(i, 1), :]


def kernel(k_cache, v_cache, input_pos, k_val, v_val):
    kc = k_cache.reshape(N_KV_HEADS, MAX_SEQ_LEN, HEAD_DIM)
    vc = v_cache.reshape(N_KV_HEADS, MAX_SEQ_LEN, HEAD_DIM)
    kv = k_val.reshape(N_KV_HEADS, Q_LEN, HEAD_DIM)
    vv = v_val.reshape(N_KV_HEADS, Q_LEN, HEAD_DIM)
    pos = input_pos.astype(jnp.int32)

    cache_spec = pl.BlockSpec((1, BS, HEAD_DIM), lambda h, s, pos_ref: (h, s, 0))
    val_spec = pl.BlockSpec((1, Q_LEN, HEAD_DIM), lambda h, s, pos_ref: (h, 0, 0))

    grid_spec = pltpu.PrefetchScalarGridSpec(
        num_scalar_prefetch=1,
        grid=(N_KV_HEADS, NB),
        in_specs=[cache_spec, cache_spec, val_spec, val_spec],
        out_specs=[cache_spec, cache_spec],
    )

    ko, vo = pl.pallas_call(
        _update_body,
        grid_spec=grid_spec,
        out_shape=[
            jax.ShapeDtypeStruct(kc.shape, kc.dtype),
            jax.ShapeDtypeStruct(vc.shape, vc.dtype),
        ],
    )(pos, kc, vc, kv, vv)

    return (ko.reshape(k_cache.shape), vo.reshape(v_cache.shape))


# speedup vs baseline: 2.8671x; 1.1057x over previous
import jax
import jax.numpy as jnp
from jax.experimental import pallas as pl
from jax.experimental.pallas import tpu as pltpu

N_KV_HEADS = 8
HEAD_DIM = 128
MAX_SEQ_LEN = 8192
Q_LEN = 16

BS = 4096
NB = MAX_SEQ_LEN // BS


def _update_body(pos_ref, kc_ref, vc_ref, kval_ref, vval_ref, ko_ref, vo_ref):
    base = pl.program_id(1) * BS
    ko_ref[...] = kc_ref[...]
    vo_ref[...] = vc_ref[...]
    for i in range(Q_LEN):
        p = pos_ref[i]
        rel = p - base

        @pl.when((p >= base) & (p < base + BS))
        def _():
            ko_ref[0, pl.ds(rel, 1), :] = kval_ref[0, pl.ds(i, 1), :]
            vo_ref[0, pl.ds(rel, 1), :] = vval_ref[0, pl.ds(i, 1), :]


def kernel(k_cache, v_cache, input_pos, k_val, v_val):
    kc = k_cache.reshape(N_KV_HEADS, MAX_SEQ_LEN, HEAD_DIM)
    vc = v_cache.reshape(N_KV_HEADS, MAX_SEQ_LEN, HEAD_DIM)
    kv = k_val.reshape(N_KV_HEADS, Q_LEN, HEAD_DIM)
    vv = v_val.reshape(N_KV_HEADS, Q_LEN, HEAD_DIM)
    pos = input_pos.astype(jnp.int32)

    cache_spec = pl.BlockSpec((1, BS, HEAD_DIM), lambda h, s, pos_ref: (h, s, 0))
    val_spec = pl.BlockSpec((1, Q_LEN, HEAD_DIM), lambda h, s, pos_ref: (h, 0, 0))

    grid_spec = pltpu.PrefetchScalarGridSpec(
        num_scalar_prefetch=1,
        grid=(N_KV_HEADS, NB),
        in_specs=[cache_spec, cache_spec, val_spec, val_spec],
        out_specs=[cache_spec, cache_spec],
    )

    ko, vo = pl.pallas_call(
        _update_body,
        grid_spec=grid_spec,
        out_shape=[
            jax.ShapeDtypeStruct(kc.shape, kc.dtype),
            jax.ShapeDtypeStruct(vc.shape, vc.dtype),
        ],
    )(pos, kc, vc, kv, vv)

    return (ko.reshape(k_cache.shape), vo.reshape(v_cache.shape))


# speedup vs baseline: 2.9446x; 1.0270x over previous
import jax
import jax.numpy as jnp
from jax.experimental import pallas as pl
from jax.experimental.pallas import tpu as pltpu

N_KV_HEADS = 8
HEAD_DIM = 128
MAX_SEQ_LEN = 8192
Q_LEN = 16

BS = 8192
NB = MAX_SEQ_LEN // BS


def _update_body(pos_ref, kc_ref, vc_ref, kval_ref, vval_ref, ko_ref, vo_ref):
    base = pl.program_id(1) * BS
    ko_ref[...] = kc_ref[...]
    vo_ref[...] = vc_ref[...]
    for i in range(Q_LEN):
        p = pos_ref[i]
        rel = p - base

        @pl.when((p >= base) & (p < base + BS))
        def _():
            ko_ref[0, pl.ds(rel, 1), :] = kval_ref[0, pl.ds(i, 1), :]
            vo_ref[0, pl.ds(rel, 1), :] = vval_ref[0, pl.ds(i, 1), :]


def kernel(k_cache, v_cache, input_pos, k_val, v_val):
    kc = k_cache.reshape(N_KV_HEADS, MAX_SEQ_LEN, HEAD_DIM)
    vc = v_cache.reshape(N_KV_HEADS, MAX_SEQ_LEN, HEAD_DIM)
    kv = k_val.reshape(N_KV_HEADS, Q_LEN, HEAD_DIM)
    vv = v_val.reshape(N_KV_HEADS, Q_LEN, HEAD_DIM)
    pos = input_pos.astype(jnp.int32)

    cache_spec = pl.BlockSpec((1, BS, HEAD_DIM), lambda h, s, pos_ref: (h, s, 0))
    val_spec = pl.BlockSpec((1, Q_LEN, HEAD_DIM), lambda h, s, pos_ref: (h, 0, 0))

    grid_spec = pltpu.PrefetchScalarGridSpec(
        num_scalar_prefetch=1,
        grid=(N_KV_HEADS, NB),
        in_specs=[cache_spec, cache_spec, val_spec, val_spec],
        out_specs=[cache_spec, cache_spec],
    )

    ko, vo = pl.pallas_call(
        _update_body,
        grid_spec=grid_spec,
        out_shape=[
            jax.ShapeDtypeStruct(kc.shape, kc.dtype),
            jax.ShapeDtypeStruct(vc.shape, vc.dtype),
        ],
    )(pos, kc, vc, kv, vv)

    return (ko.reshape(k_cache.shape), vo.reshape(v_cache.shape))
